# baseline (speedup 1.0000x reference)
import functools

import jax
import jax.numpy as jnp
import numpy as np
from jax import lax
from jax.experimental import pallas as pl
from jax.experimental.pallas import tpu as pltpu
from jax.experimental.pallas import tpu_sc as plsc

N = 10000
E = 320000
D = 128
H = 128
LABELS = 10
BN_EPS = 1e-3

NC = 2
NS = 16
NW = NC * NS

EDGES_PER_WORKER = E // NW
CHUNK = 96
HALF = CHUNK // 2
FULL_CHUNKS = EDGES_PER_WORKER // CHUNK
TAIL = EDGES_PER_WORKER - FULL_CHUNKS * CHUNK
PACK = 2 * CHUNK
TPACK = 2 * TAIL
WSTRIDE = FULL_CHUNKS * PACK + TPACK
STRIPE_BASE = 624
STRIPE_ROWS = 640
ZROWS = 16

_PERM = np.empty((D,), dtype=np.int32)
for _m in range(D // 32):
  for _k in range(16):
    _PERM[32 * _m + 2 * _k] = 32 * _m + _k
    _PERM[32 * _m + 2 * _k + 1] = 32 * _m + 16 + _k


def _to_i32_view(h_bf):
  return lax.bitcast_convert_type(h_bf.reshape(N, D // 2, 2), jnp.int32)


def _pack_edges(src, dst):
  s = src.reshape(NW, EDGES_PER_WORKER)
  d = dst.reshape(NW, EDGES_PER_WORKER)
  nf = FULL_CHUNKS * CHUNK
  full = jnp.stack([s[:, :nf].reshape(NW, FULL_CHUNKS, CHUNK),
                    d[:, :nf].reshape(NW, FULL_CHUNKS, CHUNK)], axis=2)
  tail = jnp.stack([s[:, nf:], d[:, nf:]], axis=1)
  return jnp.concatenate([full.reshape(NW, FULL_CHUNKS * PACK),
                          tail.reshape(NW, TPACK)], axis=1).reshape(-1)


def _sc_aggregate(h_bf, packed, w):
  mesh = plsc.VectorSubcoreMesh(core_axis_name="c", subcore_axis_name="s",
                                num_cores=NC, num_subcores=NS)


  @functools.partial(
      pl.kernel,
      out_type=jax.ShapeDtypeStruct((NC, N, D), jnp.float32),
      mesh=mesh,
      compiler_params=pltpu.CompilerParams(needs_layout_passes=False,
                                           use_tc_tiling_on_sc=False),
      scratch_types=[
          pltpu.VMEM((CHUNK, D // 2), jnp.int32),
          pltpu.VMEM((CHUNK, D // 2), jnp.int32),
          pltpu.VMEM((CHUNK, D), jnp.float32),
          pltpu.VMEM((CHUNK, D), jnp.float32),
          pltpu.VMEM((PACK,), jnp.int32),
          pltpu.VMEM((PACK,), jnp.int32),
          pltpu.VMEM((CHUNK,), jnp.float32),
          pltpu.VMEM((CHUNK,), jnp.float32),
          pltpu.VMEM((CHUNK,), jnp.int32),
          pltpu.VMEM((CHUNK,), jnp.int32),
          pltpu.VMEM((TAIL, D // 2), jnp.int32),
          pltpu.VMEM((TAIL, D), jnp.float32),
          pltpu.VMEM((TPACK,), jnp.int32),
          pltpu.VMEM((TAIL,), jnp.float32),
          pltpu.VMEM((TAIL,), jnp.int32),
          pltpu.VMEM((ZROWS, D), jnp.float32),
          pltpu.VMEM_SHARED((N, D), jnp.float32),
          pltpu.SemaphoreType.DMA,
          pltpu.SemaphoreType.DMA,
          pltpu.SemaphoreType.DMA,
          pltpu.SemaphoreType.DMA,
          pltpu.SemaphoreType.DMA,
          pltpu.SemaphoreType.DMA,
          pltpu.SemaphoreType.DMA,
      ],
  )
  def agg_kernel(h_hbm, p_hbm, w_hbm, out_hbm,
                 rows0, rows1, fsc0, fsc1, ib0, ib1, wb0, wb1, scb0, scb1,
                 rows_bt, fsc_t, tb, w_t, dst_t, zbuf, acc,
                 gs0, gs1, ws0, ws1, is0, is1, zsem):
    rows = [rows0, rows1]
    fsc = [fsc0, fsc1]
    ib = [ib0, ib1]
    wb = [wb0, wb1]
    scb = [scb0, scb1]
    gsem = [gs0, gs1]
    wsem = [ws0, ws1]
    isem = [is0, is1]

    core = lax.axis_index("c")
    sid = lax.axis_index("s")
    wid = core * NS + sid
    pstart = wid * WSTRIDE
    wstart = wid * EDGES_PER_WORKER

    def start_indices(i, b):
      pltpu.async_copy(p_hbm.at[pl.ds(pstart + i * PACK, PACK)], ib[b],
                       isem[b])
      pltpu.async_copy(w_hbm.at[pl.ds(wstart + i * CHUNK, CHUNK)], wb[b],
                       isem[b])

    def wait_indices(b):
      pltpu.make_async_copy(p_hbm.at[pl.ds(0, PACK)], ib[b], isem[b]).wait()
      pltpu.make_async_copy(w_hbm.at[pl.ds(0, CHUNK)], wb[b],
                            isem[b]).wait()

    start_indices(0, 0)
    start_indices(1, 1)

    zero16 = jnp.zeros((16,), jnp.float32)

    def zrow(r, carry):
      for c8 in range(D // 16):
        zbuf[r, pl.ds(c8 * 16, 16)] = zero16
      return carry

    lax.fori_loop(0, ZROWS, zrow, 0)
    for k in range(STRIPE_ROWS // ZROWS):
      pltpu.async_copy(zbuf, acc.at[pl.ds(sid * STRIPE_BASE + k * ZROWS,
                                          ZROWS)], zsem)
    wait_indices(0)
    pltpu.async_copy(h_hbm.at[ib[0].at[pl.ds(0, CHUNK)]], rows[0], gsem[0])
    for k in range(STRIPE_ROWS // ZROWS):
      pltpu.make_async_copy(zbuf, acc.at[pl.ds(sid * STRIPE_BASE + k * ZROWS,
                                               ZROWS)], zsem).wait()
    plsc.subcore_barrier()

    def scale_rows(bf_ref, row_off, out_ref, w_ref, w_off, count):
      def group(g, carry):
        w16 = w_ref[pl.ds(w_off + g * 16, 16)]
        for j in range(16):
          lane = jnp.full((16,), j, jnp.int32)
          wsplat = jnp.take_along_axis(w16, lane, axis=0,
                                       mode="promise_in_bounds")
          r = g * 16 + j
          for c in range(D // 32):
            v = plsc.bitcast(bf_ref[row_off + r, pl.ds(c * 16, 16)],
                             jnp.bfloat16)
            lo, hi = plsc.unpack(v, format=plsc.PackFormat.INTERLEAVED)
            out_ref[r, pl.ds(c * 32, 16)] = lo * wsplat
            out_ref[r, pl.ds(c * 32 + 16, 16)] = hi * wsplat
        return carry

      lax.fori_loop(0, count // 16, group, 0)

    def outer(k, carry):
      for b in range(2):
        i = k * 2 + b
        bn = (b + 1) % 2

        @pl.when(i + 1 < FULL_CHUNKS)
        def _():
          wait_indices(bn)
          pltpu.async_copy(h_hbm.at[ib[bn].at[pl.ds(0, CHUNK)]], rows[bn],
                           gsem[bn])

        pltpu.make_async_copy(h_hbm.at[ib[b].at[pl.ds(0, CHUNK)]], rows[b],
                              gsem[b]).wait()

        @pl.when(i >= 2)
        def _():
          pltpu.make_async_copy(fsc[b], acc.at[scb[b]], wsem[b]).wait()

        scale_rows(rows[b], 0, fsc[b], wb[b], 0, CHUNK)
        for g in range(CHUNK // 16):
          scb[b][pl.ds(g * 16, 16)] = ib[b][pl.ds(CHUNK + g * 16, 16)]
        pltpu.async_copy(fsc[b], acc.at[scb[b]], wsem[b], add=True)

        @pl.when(i + 2 < FULL_CHUNKS)
        def _():
          start_indices(i + 2, b)

      return carry

    lax.fori_loop(0, FULL_CHUNKS // 2, outer, 0)
    for b in range(2):
      pltpu.make_async_copy(fsc[b], acc.at[scb[b]], wsem[b]).wait()

    pltpu.sync_copy(p_hbm.at[pl.ds(pstart + FULL_CHUNKS * PACK, TPACK)], tb)
    pltpu.sync_copy(w_hbm.at[pl.ds(wstart + FULL_CHUNKS * CHUNK, TAIL)], w_t)
    dst_t[pl.ds(0, TAIL)] = tb[pl.ds(TAIL, TAIL)]
    pltpu.async_copy(h_hbm.at[tb.at[pl.ds(0, TAIL)]], rows_bt, gs0).wait()
    scale_rows(rows_bt, 0, fsc_t, w_t, 0, TAIL)
    pltpu.sync_copy(fsc_t, acc.at[dst_t], add=True)

    plsc.subcore_barrier()
    pltpu.sync_copy(acc.at[pl.ds(sid * STRIPE_BASE, STRIPE_ROWS)],
                    out_hbm.at[core, pl.ds(sid * STRIPE_BASE, STRIPE_ROWS)])

  return agg_kernel(h_bf, packed, w)


BLK = 2000
GRID = N // BLK


def _tc_mlp(h, p0, p1, W1, b1, W2, b2, gamma, beta, P):
  def body(h_ref, p0_ref, p1_ref, W1_ref, b1_ref, W2_ref, b2_ref,
           g_ref, bt_ref, P_ref, out_ref, outbf_ref):
    z = h_ref[...] + p0_ref[...] + p1_ref[...]
    a = jnp.maximum(jnp.dot(z, W1_ref[...],
                            preferred_element_type=jnp.float32)
                    + b1_ref[...], 0.0)
    b = jnp.dot(a, W2_ref[...], preferred_element_type=jnp.float32) \
        + b2_ref[...]
    hh = jnp.maximum(b * g_ref[...] + bt_ref[...], 0.0)
    out_ref[...] = hh
    outbf_ref[...] = jnp.dot(hh, P_ref[...],
                             preferred_element_type=jnp.float32
                             ).astype(jnp.bfloat16)

  row_spec = pl.BlockSpec((BLK, D), lambda i: (i, 0))
  mat_spec = lambda r, c: pl.BlockSpec((r, c), lambda i: (0, 0))
  return pl.pallas_call(
      body,
      grid=(GRID,),
      in_specs=[row_spec, row_spec, row_spec,
                mat_spec(D, H), mat_spec(1, H),
                mat_spec(H, H), mat_spec(1, H),
                mat_spec(1, H), mat_spec(1, H),
                mat_spec(H, H)],
      out_specs=[pl.BlockSpec((BLK, H), lambda i: (i, 0)),
                 pl.BlockSpec((BLK, H), lambda i: (i, 0))],
      out_shape=[jax.ShapeDtypeStruct((N, H), jnp.float32),
                 jax.ShapeDtypeStruct((N, H), jnp.bfloat16)],
  )(h, p0, p1, W1, b1, W2, b2, gamma, beta, P)


def _tc_mlp_head(h, p0, p1, W1, b1, W2, b2, gamma, beta,
                 Wm1, bm1, Wm2, bm2):
  def body(h_ref, p0_ref, p1_ref, W1_ref, b1_ref, W2_ref, b2_ref,
           g_ref, bt_ref, Wm1_ref, bm1_ref, Wm2_ref, bm2_ref, out_ref):
    z = h_ref[...] + p0_ref[...] + p1_ref[...]
    a = jnp.maximum(jnp.dot(z, W1_ref[...],
                            preferred_element_type=jnp.float32)
                    + b1_ref[...], 0.0)
    b = jnp.dot(a, W2_ref[...], preferred_element_type=jnp.float32) \
        + b2_ref[...]
    hh = jnp.maximum(b * g_ref[...] + bt_ref[...], 0.0)
    m = jnp.maximum(jnp.dot(hh, Wm1_ref[...],
                            preferred_element_type=jnp.float32)
                    + bm1_ref[...], 0.0)
    out_ref[...] = jnp.dot(m, Wm2_ref[...],
                           preferred_element_type=jnp.float32) + bm2_ref[...]

  row_spec = pl.BlockSpec((BLK, D), lambda i: (i, 0))
  mat_spec = lambda r, c: pl.BlockSpec((r, c), lambda i: (0, 0))
  return pl.pallas_call(
      body,
      grid=(GRID,),
      in_specs=[row_spec, row_spec, row_spec,
                mat_spec(D, H), mat_spec(1, H),
                mat_spec(H, H), mat_spec(1, H),
                mat_spec(1, H), mat_spec(1, H),
                mat_spec(H, 256), mat_spec(1, 256),
                mat_spec(256, LABELS), mat_spec(1, LABELS)],
      out_specs=pl.BlockSpec((BLK, LABELS), lambda i: (i, 0)),
      out_shape=jax.ShapeDtypeStruct((N, LABELS), jnp.float32),
  )(h, p0, p1, W1, b1, W2, b2, gamma, beta, Wm1, bm1, Wm2, bm2)


def kernel(x, edge_index, edge_weight, W1_0, b1_0, W2_0, b2_0, gamma_0,
           beta_0, W1_1, b1_1, W2_1, b2_1, gamma_1, beta_1, W1_2, b1_2,
           W2_2, b2_2, gamma_2, beta_2, Wm1, bm1, Wm2, bm2):
  src = edge_index[0].astype(jnp.int32)
  dst = edge_index[1].astype(jnp.int32)
  packed = _pack_edges(src, dst)

  P = jnp.eye(D, dtype=jnp.float32)[:, _PERM]
  bn_scale = 1.0 / jnp.sqrt(1.0 + BN_EPS)
  params = [(W1_0, b1_0, W2_0, b2_0, gamma_0, beta_0),
            (W1_1, b1_1, W2_1, b2_1, gamma_1, beta_1),
            (W1_2, b1_2, W2_2, b2_2, gamma_2, beta_2)]

  h = x
  h_bf = _to_i32_view(x[:, _PERM].astype(jnp.bfloat16))

  for l, (W1, b1, W2, b2, gamma, beta) in enumerate(params):
    partials = _sc_aggregate(h_bf, packed, edge_weight)
    g = (gamma * bn_scale).reshape(1, H)
    bt = beta.reshape(1, H)
    b1r = b1.reshape(1, H)
    b2r = b2.reshape(1, H)
    if l < 2:
      h, h_bf16 = _tc_mlp(h, partials[0], partials[1], W1, b1r, W2, b2r,
                          g, bt, P)
      h_bf = _to_i32_view(h_bf16)
    else:
      return _tc_mlp_head(h, partials[0], partials[1], W1, b1r, W2, b2r,
                          g, bt, Wm1, bm1.reshape(1, 256), Wm2,
                          bm2.reshape(1, LABELS))


# speedup vs baseline: 2.2802x; 2.2802x over previous
import functools

import jax
import jax.numpy as jnp
from jax import lax
from jax.experimental import pallas as pl
from jax.experimental.pallas import tpu as pltpu
from jax.experimental.pallas import tpu_sc as plsc

N = 10000
E = 320000
D = 128
H = 128
LABELS = 10
BN_EPS = 1e-3

NC = 2
NS = 16
NW = NC * NS

EDGES_PER_WORKER = E // NW
CHUNK = 128
FULL_CHUNKS = EDGES_PER_WORKER // CHUNK
TAIL = EDGES_PER_WORKER - FULL_CHUNKS * CHUNK
PACK = 2 * CHUNK
TPACK = 2 * TAIL
WSTRIDE = FULL_CHUNKS * PACK + TPACK
STRIPE_BASE = 624
STRIPE_ROWS = 640
ZROWS = 64


def _pack_edges(src, dst):
  s = src.reshape(NW, EDGES_PER_WORKER)
  d = dst.reshape(NW, EDGES_PER_WORKER)
  nf = FULL_CHUNKS * CHUNK
  full = jnp.stack([s[:, :nf].reshape(NW, FULL_CHUNKS, CHUNK),
                    d[:, :nf].reshape(NW, FULL_CHUNKS, CHUNK)], axis=2)
  tail = jnp.stack([s[:, nf:], d[:, nf:]], axis=1)
  return jnp.concatenate([full.reshape(NW, FULL_CHUNKS * PACK),
                          tail.reshape(NW, TPACK)], axis=1).reshape(-1)


def _sc_aggregate(h, packed, w):
  mesh = plsc.VectorSubcoreMesh(core_axis_name="c", subcore_axis_name="s",
                                num_cores=NC, num_subcores=NS)


  @functools.partial(
      pl.kernel,
      out_type=jax.ShapeDtypeStruct((NC, N, D), jnp.float32),
      mesh=mesh,
      scratch_types=[
          pltpu.VMEM((CHUNK, D), jnp.float32),
          pltpu.VMEM((CHUNK, D), jnp.float32),
          pltpu.VMEM((PACK,), jnp.int32),
          pltpu.VMEM((PACK,), jnp.int32),
          pltpu.VMEM((CHUNK,), jnp.float32),
          pltpu.VMEM((CHUNK,), jnp.float32),
          pltpu.VMEM((CHUNK,), jnp.int32),
          pltpu.VMEM((CHUNK,), jnp.int32),
          pltpu.VMEM((TAIL, D), jnp.float32),
          pltpu.VMEM((TPACK,), jnp.int32),
          pltpu.VMEM((TAIL,), jnp.float32),
          pltpu.VMEM((TAIL,), jnp.int32),
          pltpu.VMEM((ZROWS, D), jnp.float32),
          pltpu.VMEM_SHARED((N, D), jnp.float32),
          pltpu.SemaphoreType.DMA,
          pltpu.SemaphoreType.DMA,
          pltpu.SemaphoreType.DMA,
          pltpu.SemaphoreType.DMA,
          pltpu.SemaphoreType.DMA,
          pltpu.SemaphoreType.DMA,
          pltpu.SemaphoreType.DMA,
      ],
  )
  def agg_kernel(h_hbm, p_hbm, w_hbm, out_hbm,
                 rows0, rows1, ib0, ib1, wb0, wb1, scb0, scb1,
                 rows_t, tb, w_t, dst_t, zbuf, acc,
                 gs0, gs1, ws0, ws1, is0, is1, zsem):
    rows = [rows0, rows1]
    ib = [ib0, ib1]
    wb = [wb0, wb1]
    scb = [scb0, scb1]
    gsem = [gs0, gs1]
    wsem = [ws0, ws1]
    isem = [is0, is1]

    core = lax.axis_index("c")
    sid = lax.axis_index("s")
    wid = core * NS + sid
    pstart = wid * WSTRIDE
    wstart = wid * EDGES_PER_WORKER

    def start_indices(i, b):
      pltpu.async_copy(p_hbm.at[pl.ds(pstart + i * PACK, PACK)], ib[b],
                       isem[b])
      pltpu.async_copy(w_hbm.at[pl.ds(wstart + i * CHUNK, CHUNK)], wb[b],
                       isem[b])

    def wait_indices(b):
      pltpu.make_async_copy(p_hbm.at[pl.ds(0, PACK)], ib[b], isem[b]).wait()
      pltpu.make_async_copy(w_hbm.at[pl.ds(0, CHUNK)], wb[b],
                            isem[b]).wait()

    start_indices(0, 0)
    start_indices(1, 1)

    zero16 = jnp.zeros((16,), jnp.float32)

    def zrow(r, carry):
      for c8 in range(D // 16):
        zbuf[r, pl.ds(c8 * 16, 16)] = zero16
      return carry

    lax.fori_loop(0, ZROWS, zrow, 0)
    for k in range(STRIPE_ROWS // ZROWS):
      pltpu.async_copy(zbuf, acc.at[pl.ds(sid * STRIPE_BASE + k * ZROWS,
                                          ZROWS)], zsem)
    wait_indices(0)
    pltpu.async_copy(h_hbm.at[ib[0].at[pl.ds(0, CHUNK)]], rows[0], gsem[0])
    for k in range(STRIPE_ROWS // ZROWS):
      pltpu.make_async_copy(zbuf, acc.at[pl.ds(sid * STRIPE_BASE + k * ZROWS,
                                               ZROWS)], zsem).wait()
    plsc.subcore_barrier()

    def scale_rows(rows_ref, w_ref, count):
      def group(g, carry):
        w16 = w_ref[pl.ds(g * 16, 16)]
        for j in range(16):
          lane = jnp.full((16,), j, jnp.int32)
          wsplat = jnp.take_along_axis(w16, lane, axis=0,
                                       mode="promise_in_bounds")
          r = g * 16 + j
          for c8 in range(D // 16):
            sl = pl.ds(c8 * 16, 16)
            rows_ref[r, sl] = rows_ref[r, sl] * wsplat
        return carry

      lax.fori_loop(0, count // 16, group, 0)

    def outer(k, carry):
      for b in range(2):
        i = k * 2 + b
        bn = (b + 1) % 2

        @pl.when(i + 1 < FULL_CHUNKS)
        def _():
          wait_indices(bn)

        @pl.when(i >= 1)
        def _():
          pltpu.make_async_copy(rows[bn], acc.at[scb[bn]], wsem[bn]).wait()

        @pl.when(i + 1 < FULL_CHUNKS)
        def _():
          pltpu.async_copy(h_hbm.at[ib[bn].at[pl.ds(0, CHUNK)]], rows[bn],
                           gsem[bn])

        pltpu.make_async_copy(h_hbm.at[ib[b].at[pl.ds(0, CHUNK)]], rows[b],
                              gsem[b]).wait()
        scale_rows(rows[b], wb[b], CHUNK)
        for g in range(CHUNK // 16):
          scb[b][pl.ds(g * 16, 16)] = ib[b][pl.ds(CHUNK + g * 16, 16)]
        pltpu.async_copy(rows[b], acc.at[scb[b]], wsem[b], add=True)

        @pl.when(i + 2 < FULL_CHUNKS)
        def _():
          start_indices(i + 2, b)

      return carry

    lax.fori_loop(0, FULL_CHUNKS // 2, outer, 0)
    pltpu.make_async_copy(rows[1], acc.at[scb[1]], wsem[1]).wait()

    pltpu.sync_copy(p_hbm.at[pl.ds(pstart + FULL_CHUNKS * PACK, TPACK)], tb)
    pltpu.sync_copy(w_hbm.at[pl.ds(wstart + FULL_CHUNKS * CHUNK, TAIL)], w_t)
    dst_t[pl.ds(0, TAIL)] = tb[pl.ds(TAIL, TAIL)]
    pltpu.async_copy(h_hbm.at[tb.at[pl.ds(0, TAIL)]], rows_t, gs0).wait()
    scale_rows(rows_t, w_t, TAIL)
    pltpu.sync_copy(rows_t, acc.at[dst_t], add=True)

    plsc.subcore_barrier()
    pltpu.sync_copy(acc.at[pl.ds(sid * STRIPE_BASE, STRIPE_ROWS)],
                    out_hbm.at[core, pl.ds(sid * STRIPE_BASE, STRIPE_ROWS)])

  return agg_kernel(h, packed, w)


BLK = 2000
GRID = N // BLK


def _tc_mlp(h, p0, p1, W1, b1, W2, b2, gamma, beta):
  def body(h_ref, p0_ref, p1_ref, W1_ref, b1_ref, W2_ref, b2_ref,
           g_ref, bt_ref, out_ref):
    z = h_ref[...] + p0_ref[...] + p1_ref[...]
    a = jnp.maximum(jnp.dot(z, W1_ref[...],
                            preferred_element_type=jnp.float32)
                    + b1_ref[...], 0.0)
    b = jnp.dot(a, W2_ref[...], preferred_element_type=jnp.float32) \
        + b2_ref[...]
    out_ref[...] = jnp.maximum(b * g_ref[...] + bt_ref[...], 0.0)

  row_spec = pl.BlockSpec((BLK, D), lambda i: (i, 0))
  return pl.pallas_call(
      body,
      grid=(GRID,),
      in_specs=[row_spec, row_spec, row_spec,
                pl.BlockSpec((D, H), lambda i: (0, 0)),
                pl.BlockSpec((1, H), lambda i: (0, 0)),
                pl.BlockSpec((H, H), lambda i: (0, 0)),
                pl.BlockSpec((1, H), lambda i: (0, 0)),
                pl.BlockSpec((1, H), lambda i: (0, 0)),
                pl.BlockSpec((1, H), lambda i: (0, 0))],
      out_specs=pl.BlockSpec((BLK, H), lambda i: (i, 0)),
      out_shape=jax.ShapeDtypeStruct((N, H), jnp.float32),
  )(h, p0, p1, W1, b1, W2, b2, gamma, beta)


def _tc_mlp_head(h, p0, p1, W1, b1, W2, b2, gamma, beta,
                 Wm1, bm1, Wm2, bm2):
  def body(h_ref, p0_ref, p1_ref, W1_ref, b1_ref, W2_ref, b2_ref,
           g_ref, bt_ref, Wm1_ref, bm1_ref, Wm2_ref, bm2_ref, out_ref):
    z = h_ref[...] + p0_ref[...] + p1_ref[...]
    a = jnp.maximum(jnp.dot(z, W1_ref[...],
                            preferred_element_type=jnp.float32)
                    + b1_ref[...], 0.0)
    b = jnp.dot(a, W2_ref[...], preferred_element_type=jnp.float32) \
        + b2_ref[...]
    hh = jnp.maximum(b * g_ref[...] + bt_ref[...], 0.0)
    m = jnp.maximum(jnp.dot(hh, Wm1_ref[...],
                            preferred_element_type=jnp.float32)
                    + bm1_ref[...], 0.0)
    out_ref[...] = jnp.dot(m, Wm2_ref[...],
                           preferred_element_type=jnp.float32) + bm2_ref[...]

  row_spec = pl.BlockSpec((BLK, D), lambda i: (i, 0))
  return pl.pallas_call(
      body,
      grid=(GRID,),
      in_specs=[row_spec, row_spec, row_spec,
                pl.BlockSpec((D, H), lambda i: (0, 0)),
                pl.BlockSpec((1, H), lambda i: (0, 0)),
                pl.BlockSpec((H, H), lambda i: (0, 0)),
                pl.BlockSpec((1, H), lambda i: (0, 0)),
                pl.BlockSpec((1, H), lambda i: (0, 0)),
                pl.BlockSpec((1, H), lambda i: (0, 0)),
                pl.BlockSpec((H, 256), lambda i: (0, 0)),
                pl.BlockSpec((1, 256), lambda i: (0, 0)),
                pl.BlockSpec((256, LABELS), lambda i: (0, 0)),
                pl.BlockSpec((1, LABELS), lambda i: (0, 0))],
      out_specs=pl.BlockSpec((BLK, LABELS), lambda i: (i, 0)),
      out_shape=jax.ShapeDtypeStruct((N, LABELS), jnp.float32),
  )(h, p0, p1, W1, b1, W2, b2, gamma, beta, Wm1, bm1, Wm2, bm2)


def kernel(x, edge_index, edge_weight, W1_0, b1_0, W2_0, b2_0, gamma_0,
           beta_0, W1_1, b1_1, W2_1, b2_1, gamma_1, beta_1, W1_2, b1_2,
           W2_2, b2_2, gamma_2, beta_2, Wm1, bm1, Wm2, bm2):
  src = edge_index[0].astype(jnp.int32)
  dst = edge_index[1].astype(jnp.int32)
  packed = _pack_edges(src, dst)

  bn_scale = 1.0 / jnp.sqrt(1.0 + BN_EPS)
  params = [(W1_0, b1_0, W2_0, b2_0, gamma_0, beta_0),
            (W1_1, b1_1, W2_1, b2_1, gamma_1, beta_1),
            (W1_2, b1_2, W2_2, b2_2, gamma_2, beta_2)]

  h = x
  for l, (W1, b1, W2, b2, gamma, beta) in enumerate(params):
    partials = _sc_aggregate(h, packed, edge_weight)
    g = (gamma * bn_scale).reshape(1, H)
    bt = beta.reshape(1, H)
    b1r = b1.reshape(1, H)
    b2r = b2.reshape(1, H)
    if l < 2:
      h = _tc_mlp(h, partials[0], partials[1], W1, b1r, W2, b2r, g, bt)
    else:
      return _tc_mlp_head(h, partials[0], partials[1], W1, b1r, W2, b2r,
                          g, bt, Wm1, bm1.reshape(1, 256), Wm2,
                          bm2.reshape(1, LABELS))
